# Initial kernel scaffold; baseline (speedup 1.0000x reference)
import jax
import jax.numpy as jnp
from jax.experimental import pallas as pl
from jax.experimental.pallas import tpu as pltpu

N_NODES = 1365
N_CLASSES = 1024
DEPTH = 5
FEAT = 1024
TEMP = 0.2
BATCH = 4096
NPAD = 1408
BB = 512
_OFFSETS = (1, 5, 21, 85, 341)
_SUM_GJ = float(sum(range(DEPTH)))


def _loss_kernel(labels_ref, box_ref, memT_ref, out_ref):
    i = pl.program_id(0)
    lab = labels_ref[0]
    box = box_ref[...]
    logits = jnp.dot(box, memT_ref[...], preferred_element_type=jnp.float32)
    t = logits * (1.0 / TEMP)
    col = jax.lax.broadcasted_iota(jnp.int32, (BB, NPAD), 1)
    valid_col = col < N_NODES
    m = jnp.max(jnp.where(valid_col, t, -jnp.inf), axis=1, keepdims=True)
    denom = jnp.sum(jnp.where(valid_col, jnp.exp(t - m), 0.0),
                    axis=1, keepdims=True)
    log_denom = m + jnp.log(denom)

    mask = lab != N_CLASSES
    safe = jnp.where(mask, lab, 0)
    num = jnp.zeros((BB, 1), jnp.float32)
    for d in range(1, DEPTH):
        idx = _OFFSETS[d] + safe // (4 ** (DEPTH - 1 - d))
        num += float(d) * jnp.sum(jnp.where(col == idx, t, 0.0),
                                  axis=1, keepdims=True)
    per_sample = log_denom - num * (1.0 / _SUM_GJ)
    part = jnp.sum(jnp.where(mask, per_sample, 0.0)) * 0.001

    @pl.when(i == 0)
    def _init():
        out_ref[0, 0] = 0.0
    out_ref[0, 0] += part


def kernel(gt_labels, box_features, memory, trace_table):
    del trace_table
    nb = BATCH // BB
    labels3 = gt_labels.astype(jnp.int32).reshape(nb, BB, 1)
    memT = jnp.pad(memory, ((0, NPAD - N_NODES), (0, 0))).T
    out = pl.pallas_call(
        _loss_kernel,
        grid=(nb,),
        in_specs=[
            pl.BlockSpec((1, BB, 1), lambda i: (i, 0, 0)),
            pl.BlockSpec((BB, FEAT), lambda i: (i, 0)),
            pl.BlockSpec((FEAT, NPAD), lambda i: (0, 0)),
        ],
        out_specs=pl.BlockSpec((1, 1), lambda i: (0, 0)),
        out_shape=jax.ShapeDtypeStruct((1, 1), jnp.float32),
    )(labels3, box_features, memT)
    return out[0, 0]


# speedup vs baseline: 1.7591x; 1.7591x over previous
import jax
import jax.numpy as jnp
from jax.experimental import pallas as pl
from jax.experimental.pallas import tpu as pltpu

N_NODES = 1365
N_CLASSES = 1024
DEPTH = 5
FEAT = 1024
TEMP = 0.2
BATCH = 4096
NPAD = 1408
BB = 512
_OFFSETS = (1, 5, 21, 85, 341)
_SUM_GJ = float(sum(range(DEPTH)))


def _loss_kernel(labels_ref, box_ref, memT_ref, out_ref):
    i = pl.program_id(0)
    lab = labels_ref[0]
    box = box_ref[...]
    logits = jnp.dot(box, memT_ref[...], preferred_element_type=jnp.float32)
    t = logits * (1.0 / TEMP)
    col = jax.lax.broadcasted_iota(jnp.int32, (BB, NPAD), 1)
    valid_col = col < N_NODES
    m = jnp.max(jnp.where(valid_col, t, -jnp.inf), axis=1, keepdims=True)
    denom = jnp.sum(jnp.where(valid_col, jnp.exp(t - m), 0.0),
                    axis=1, keepdims=True)
    log_denom = m + jnp.log(denom)

    mask = lab != N_CLASSES
    safe = jnp.where(mask, lab, 0)
    num = jnp.zeros((BB, 1), jnp.float32)
    for d in range(1, DEPTH):
        idx = _OFFSETS[d] + safe // (4 ** (DEPTH - 1 - d))
        num += float(d) * jnp.sum(jnp.where(col == idx, t, 0.0),
                                  axis=1, keepdims=True)
    per_sample = log_denom - num * (1.0 / _SUM_GJ)
    part = (jnp.sum(jnp.where(mask, per_sample, 0.0)) * 0.001).reshape(1, 1)

    @pl.when(i == 0)
    def _init():
        out_ref[...] = jnp.zeros((1, 1), jnp.float32)
    out_ref[...] += part


def kernel(gt_labels, box_features, memory, trace_table):
    del trace_table
    nb = BATCH // BB
    labels3 = gt_labels.astype(jnp.int32).reshape(nb, BB, 1)
    memT = jnp.pad(memory, ((0, NPAD - N_NODES), (0, 0))).T
    out = pl.pallas_call(
        _loss_kernel,
        grid=(nb,),
        in_specs=[
            pl.BlockSpec((1, BB, 1), lambda i: (i, 0, 0)),
            pl.BlockSpec((BB, FEAT), lambda i: (i, 0)),
            pl.BlockSpec((FEAT, NPAD), lambda i: (0, 0)),
        ],
        out_specs=pl.BlockSpec((1, 1), lambda i: (0, 0)),
        out_shape=jax.ShapeDtypeStruct((1, 1), jnp.float32),
    )(labels3, box_features, memT)
    return out[0, 0]
